# baseline (speedup 1.0000x reference)
import jax
import jax.numpy as jnp
from jax import lax
from jax.experimental import pallas as pl
from jax.experimental.pallas import tpu as pltpu

N = 2048
BI = 512
NC = N // BI
NBH = 12


def _split2(m):
    l0 = m.astype(jnp.bfloat16)
    r1 = m - l0.astype(jnp.float32)
    l1 = r1.astype(jnp.bfloat16)
    return (l0, l1)


def _mm2(lhs_parts, rhs_b):
    dims = (((1,), (0,)), ((), ()))
    out = None
    for lp in lhs_parts:
        t = jax.lax.dot_general(lp, rhs_b, dims,
                                preferred_element_type=jnp.float32)
        out = t if out is None else out + t
    return out


def _rank_pv(row_ref, col_ref):
    x_row = row_ref[0]
    k_row = x_row * 100.0
    acc = jnp.zeros((2, N), jnp.float32)

    for a in range(NC):
        w = N - (a + 1) * BI
        kc = col_ref[0, a * BI:(a + 1) * BI, :] * 100.0
        xa = row_ref[0, :, a * BI:(a + 1) * BI]
        lhs_a = _split2(jnp.concatenate(
            [jnp.ones((1, BI), jnp.float32), xa], axis=0))

        kra = k_row[:, a * BI:(a + 1) * BI]
        il = lax.broadcasted_iota(jnp.int32, (BI, BI), 0)
        jl = lax.broadcasted_iota(jnp.int32, (BI, BI), 1)
        cd = (kc > kra) | ((kc == kra) & (il < jl))
        upd = _mm2(lhs_a, cd.astype(jnp.bfloat16))

        pieces = []
        if a > 0:
            pieces.append(jnp.zeros((2, a * BI), jnp.float32))
        if w > 0:
            krl = k_row[:, (a + 1) * BI:]
            xl = x_row[:, (a + 1) * BI:]
            gb = (kc >= krl).astype(jnp.bfloat16)
            rowside = _mm2(lhs_a, gb)
            gt = lax.transpose(gb, (1, 0))
            lhs_l = _split2(jnp.concatenate(
                [jnp.ones((1, w), jnp.float32), xl], axis=0))
            res = _mm2(lhs_l, gt)
            xsum = jnp.sum(xl)
            upd = upd + jnp.concatenate(
                [jnp.float32(w) - res[0:1, :],
                 xsum - res[1:2, :]], axis=0)
            pieces.append(upd)
            pieces.append(rowside)
        else:
            pieces.append(upd)
        acc = acc + jnp.concatenate(pieces, axis=1)

    rank_f = acc[0:1, :]
    s = acc[1:2, :] + x_row
    rr = s - (rank_f + 1.0) * x_row
    rank_h = jnp.clip(rr, 0.0, rank_f)
    pv = jnp.exp(-10.0 * (rank_f - rank_h))
    return jnp.concatenate([rank_f, pv], axis=0)


def _perm_kernel(rn_ref, cn_ref, r0_ref, c0_ref, out_ref, bankA, bankB):
    i = pl.program_id(0)

    @pl.when(i == 0)
    def _prologue():
        bankA[...] = _rank_pv(r0_ref, c0_ref)

    @pl.when(i < NBH - 1)
    def _compute_next():
        final = _rank_pv(rn_ref, cn_ref)

        @pl.when(i % 2 == 1)
        def _wa():
            bankA[...] = final

        @pl.when(i % 2 == 0)
        def _wb():
            bankB[...] = final

    va = bankA[...]
    vb = bankB[...]
    sel = jnp.where(i % 2 == 0, va, vb)
    rank_i = sel[0:1, :].astype(jnp.int32)
    pv_row = sel[1:2, :]
    rows = lax.broadcasted_iota(jnp.int32, (N, N), 0)
    out_ref[0] = jnp.where(rank_i == rows, pv_row, 0.0)


@jax.jit
def kernel(ranking):
    b_s, h, node_num = ranking.shape[:3]
    bh = b_s * h
    r2 = ranking.reshape(bh, 1, node_num)
    r3 = ranking.reshape(bh, node_num, 1)
    return pl.pallas_call(
        _perm_kernel,
        grid=(bh,),
        in_specs=[
            pl.BlockSpec((1, 1, node_num),
                         lambda i: (jnp.minimum(i + 1, NBH - 1), 0, 0)),
            pl.BlockSpec((1, node_num, 1),
                         lambda i: (jnp.minimum(i + 1, NBH - 1), 0, 0)),
            pl.BlockSpec((1, 1, node_num), lambda i: (0, 0, 0)),
            pl.BlockSpec((1, node_num, 1), lambda i: (0, 0, 0)),
        ],
        out_specs=pl.BlockSpec((1, node_num, node_num), lambda i: (i, 0, 0)),
        out_shape=jax.ShapeDtypeStruct((bh, node_num, node_num), jnp.float32),
        scratch_shapes=[
            pltpu.VMEM((2, node_num), jnp.float32),
            pltpu.VMEM((2, node_num), jnp.float32),
        ],
    )(r2, r3, r2, r3)


# speedup vs baseline: 1.0096x; 1.0096x over previous
import jax
import jax.numpy as jnp
from jax import lax
from jax.experimental import pallas as pl
from jax.experimental.pallas import tpu as pltpu

N = 2048
BI = 256
NC = N // BI
NBH = 12


def _split2(m):
    l0 = m.astype(jnp.bfloat16)
    r1 = m - l0.astype(jnp.float32)
    l1 = r1.astype(jnp.bfloat16)
    return (l0, l1)


def _mm2(lhs_parts, rhs_b):
    dims = (((1,), (0,)), ((), ()))
    out = None
    for lp in lhs_parts:
        t = jax.lax.dot_general(lp, rhs_b, dims,
                                preferred_element_type=jnp.float32)
        out = t if out is None else out + t
    return out


def _rank_pv(row_ref, col_ref):
    x_row = row_ref[0]
    k_row = x_row * 100.0
    acc = jnp.zeros((2, N), jnp.float32)

    for a in range(NC):
        w = N - (a + 1) * BI
        kc = col_ref[0, a * BI:(a + 1) * BI, :] * 100.0
        xa = row_ref[0, :, a * BI:(a + 1) * BI]
        lhs_a = _split2(jnp.concatenate(
            [jnp.ones((1, BI), jnp.float32), xa], axis=0))

        kra = k_row[:, a * BI:(a + 1) * BI]
        il = lax.broadcasted_iota(jnp.int32, (BI, BI), 0)
        jl = lax.broadcasted_iota(jnp.int32, (BI, BI), 1)
        cd = (kc > kra) | ((kc == kra) & (il < jl))
        upd = _mm2(lhs_a, cd.astype(jnp.bfloat16))

        pieces = []
        if a > 0:
            pieces.append(jnp.zeros((2, a * BI), jnp.float32))
        if w > 0:
            krl = k_row[:, (a + 1) * BI:]
            xl = x_row[:, (a + 1) * BI:]
            gb = (kc >= krl).astype(jnp.bfloat16)
            rowside = _mm2(lhs_a, gb)
            gt = lax.transpose(gb, (1, 0))
            lhs_l = _split2(jnp.concatenate(
                [jnp.ones((1, w), jnp.float32), xl], axis=0))
            res = _mm2(lhs_l, gt)
            xsum = jnp.sum(xl)
            upd = upd + jnp.concatenate(
                [jnp.float32(w) - res[0:1, :],
                 xsum - res[1:2, :]], axis=0)
            pieces.append(upd)
            pieces.append(rowside)
        else:
            pieces.append(upd)
        acc = acc + jnp.concatenate(pieces, axis=1)

    rank_f = acc[0:1, :]
    s = acc[1:2, :] + x_row
    rr = s - (rank_f + 1.0) * x_row
    rank_h = jnp.clip(rr, 0.0, rank_f)
    pv = jnp.exp(-10.0 * (rank_f - rank_h))
    return jnp.concatenate([rank_f, pv], axis=0)


def _perm_kernel(rn_ref, cn_ref, r0_ref, c0_ref, out_ref, bankA, bankB):
    i = pl.program_id(0)

    @pl.when(i == 0)
    def _prologue():
        bankA[...] = _rank_pv(r0_ref, c0_ref)

    @pl.when(i < NBH - 1)
    def _compute_next():
        final = _rank_pv(rn_ref, cn_ref)

        @pl.when(i % 2 == 1)
        def _wa():
            bankA[...] = final

        @pl.when(i % 2 == 0)
        def _wb():
            bankB[...] = final

    va = bankA[...]
    vb = bankB[...]
    sel = jnp.where(i % 2 == 0, va, vb)
    rank_i = sel[0:1, :].astype(jnp.int32)
    pv_row = sel[1:2, :]
    rows = lax.broadcasted_iota(jnp.int32, (N, N), 0)
    out_ref[0] = jnp.where(rank_i == rows, pv_row, 0.0)


@jax.jit
def kernel(ranking):
    b_s, h, node_num = ranking.shape[:3]
    bh = b_s * h
    r2 = ranking.reshape(bh, 1, node_num)
    r3 = ranking.reshape(bh, node_num, 1)
    return pl.pallas_call(
        _perm_kernel,
        grid=(bh,),
        in_specs=[
            pl.BlockSpec((1, 1, node_num),
                         lambda i: (jnp.minimum(i + 1, NBH - 1), 0, 0)),
            pl.BlockSpec((1, node_num, 1),
                         lambda i: (jnp.minimum(i + 1, NBH - 1), 0, 0)),
            pl.BlockSpec((1, 1, node_num), lambda i: (0, 0, 0)),
            pl.BlockSpec((1, node_num, 1), lambda i: (0, 0, 0)),
        ],
        out_specs=pl.BlockSpec((1, node_num, node_num), lambda i: (i, 0, 0)),
        out_shape=jax.ShapeDtypeStruct((bh, node_num, node_num), jnp.float32),
        scratch_shapes=[
            pltpu.VMEM((2, node_num), jnp.float32),
            pltpu.VMEM((2, node_num), jnp.float32),
        ],
    )(r2, r3, r2, r3)
